# Initial kernel scaffold; baseline (speedup 1.0000x reference)
import jax
import jax.numpy as jnp
from jax.experimental import pallas as pl


def kernel(x, table):
    raise NotImplementedError("write your pallas kernel here")


# speedup vs baseline: 1.8250x; 1.8250x over previous
import functools

import jax
import jax.numpy as jnp
from jax import lax
from jax.experimental import pallas as pl
from jax.experimental.pallas import tpu as pltpu
from jax.experimental.pallas import tpu_sc as plsc

N_EMBED = 1024
SCALE = 32.0

_info = plsc.get_sparse_core_info()
_NC, _NS = _info.num_cores, _info.num_subcores
_NW = _NC * _NS


def _scale_table(table):
    blk = 512

    def body(t_ref, o_ref):
        o_ref[...] = t_ref[...] * SCALE

    return pl.pallas_call(
        body,
        out_shape=jax.ShapeDtypeStruct(table.shape, table.dtype),
        grid=(table.shape[0] // blk,),
        in_specs=[pl.BlockSpec((blk, table.shape[1]), lambda i: (i, 0))],
        out_specs=pl.BlockSpec((blk, table.shape[1]), lambda i: (i, 0)),
    )(table)


@functools.cache
def _make_gather(B, D):
    per_w = B // _NW
    K = 64
    n_chunks = per_w // K
    mesh = plsc.VectorSubcoreMesh(core_axis_name="c", subcore_axis_name="s")

    @functools.partial(
        pl.kernel,
        mesh=mesh,
        out_type=jax.ShapeDtypeStruct((B, D), jnp.float32),
        scratch_types=[
            pltpu.VMEM((per_w,), jnp.int32),
            pltpu.VMEM((K, D), jnp.float32),
            pltpu.SemaphoreType.DMA,
            pltpu.SemaphoreType.DMA,
        ],
    )
    def gather_kernel(table_hbm, idx_hbm, out_hbm, idx_v, rows_v, gsem, ssem):
        wid = lax.axis_index("s") * _NC + lax.axis_index("c")
        base = wid * per_w
        pltpu.sync_copy(idx_hbm.at[pl.ds(base, per_w)], idx_v)

        def step(c, carry):
            g = pltpu.make_async_copy(
                table_hbm.at[idx_v.at[pl.ds(c * K, K)]], rows_v, gsem)
            g.start()
            g.wait()
            s = pltpu.make_async_copy(
                rows_v, out_hbm.at[pl.ds(base + c * K, K)], ssem)
            s.start()
            s.wait()
            return carry

        lax.fori_loop(0, n_chunks, step, 0)

    return gather_kernel


def kernel(x, table):
    B, S = x.shape
    _, D = table.shape
    scaled = _scale_table(table)
    idx = x.reshape(B * S).astype(jnp.int32)
    out = _make_gather(B * S, D)(scaled, idx)
    return out.reshape(B, S, D)


# speedup vs baseline: 1.9708x; 1.0799x over previous
import functools

import jax
import jax.numpy as jnp
from jax import lax
from jax.experimental import pallas as pl
from jax.experimental.pallas import tpu as pltpu
from jax.experimental.pallas import tpu_sc as plsc

N_EMBED = 1024
SCALE = 32.0

_info = plsc.get_sparse_core_info()
_NC, _NS = _info.num_cores, _info.num_subcores
_NW = _NC * _NS


def _scale_table(table):
    blk = 512

    def body(t_ref, o_ref):
        o_ref[...] = t_ref[...] * SCALE

    return pl.pallas_call(
        body,
        out_shape=jax.ShapeDtypeStruct(table.shape, table.dtype),
        grid=(table.shape[0] // blk,),
        in_specs=[pl.BlockSpec((blk, table.shape[1]), lambda i: (i, 0))],
        out_specs=pl.BlockSpec((blk, table.shape[1]), lambda i: (i, 0)),
    )(table)


@functools.cache
def _make_gather(B, D):
    per_w = B // _NW
    K = 32
    n_chunks = per_w // K
    mesh = plsc.VectorSubcoreMesh(core_axis_name="c", subcore_axis_name="s")

    @functools.partial(
        pl.kernel,
        mesh=mesh,
        out_type=jax.ShapeDtypeStruct((B, D), jnp.float32),
        scratch_types=[
            pltpu.VMEM((per_w,), jnp.int32),
            pltpu.VMEM((2, K, D), jnp.float32),
            (pltpu.SemaphoreType.DMA, pltpu.SemaphoreType.DMA),
            (pltpu.SemaphoreType.DMA, pltpu.SemaphoreType.DMA),
        ],
    )
    def gather_kernel(table_hbm, idx_hbm, out_hbm, idx_v, rows_v, gsems, ssems):
        wid = lax.axis_index("s") * _NC + lax.axis_index("c")
        base = wid * per_w
        pltpu.sync_copy(idx_hbm.at[pl.ds(base, per_w)], idx_v)

        def G(i, b):
            return pltpu.make_async_copy(
                table_hbm.at[idx_v.at[pl.ds(i * K, K)]], rows_v.at[b], gsems[b])

        def S(i, b):
            return pltpu.make_async_copy(
                rows_v.at[b], out_hbm.at[pl.ds(base + i * K, K)], ssems[b])

        G(0, 0).start()
        G(1, 1).start()
        S_0 = S(0, 0)
        G(0, 0).wait()
        S_0.start()

        @pl.loop(0, (n_chunks - 2) // 2)
        def _pair(j):
            c = 1 + 2 * j
            S(c - 1, 0).wait()
            G(c + 1, 0).start()
            G(c, 1).wait()
            S(c, 1).start()

            S(c, 1).wait()
            G(c + 2, 1).start()
            G(c + 1, 0).wait()
            S(c + 1, 0).start()

        c_last = n_chunks - 1
        S(c_last - 1, 0).wait()
        G(c_last, 1).wait()
        S(c_last, 1).start()
        S(c_last, 1).wait()

    return gather_kernel


def kernel(x, table):
    B, S = x.shape
    _, D = table.shape
    scaled = _scale_table(table)
    idx = x.reshape(B * S).astype(jnp.int32)
    out = _make_gather(B * S, D)(scaled, idx)
    return out.reshape(B, S, D)


# speedup vs baseline: 2.0710x; 1.0508x over previous
import functools

import jax
import jax.numpy as jnp
from jax import lax
from jax.experimental import pallas as pl
from jax.experimental.pallas import tpu as pltpu
from jax.experimental.pallas import tpu_sc as plsc

N_EMBED = 1024
SCALE = 32.0

_info = plsc.get_sparse_core_info()
_NC, _NS = _info.num_cores, _info.num_subcores
_NW = _NC * _NS


@functools.cache
def _make_gather(B, D):
    per_w = B // _NW
    K = 32
    n_chunks = per_w // K
    mesh = plsc.VectorSubcoreMesh(core_axis_name="c", subcore_axis_name="s")

    @functools.partial(
        pl.kernel,
        mesh=mesh,
        out_type=jax.ShapeDtypeStruct((B, D), jnp.float32),
        scratch_types=[
            pltpu.VMEM((per_w,), jnp.int32),
            pltpu.VMEM((2, K, D), jnp.float32),
            (pltpu.SemaphoreType.DMA, pltpu.SemaphoreType.DMA),
            (pltpu.SemaphoreType.DMA, pltpu.SemaphoreType.DMA),
        ],
    )
    def gather_kernel(table_hbm, idx_hbm, out_hbm, idx_v, rows_v, gsems, ssems):
        wid = lax.axis_index("s") * _NC + lax.axis_index("c")
        base = wid * per_w
        pltpu.sync_copy(idx_hbm.at[pl.ds(base, per_w)], idx_v)

        def G(i, b):
            return pltpu.make_async_copy(
                table_hbm.at[idx_v.at[pl.ds(i * K, K)]], rows_v.at[b], gsems[b])

        def S(i, b):
            return pltpu.make_async_copy(
                rows_v.at[b], out_hbm.at[pl.ds(base + i * K, K)], ssems[b])

        def scale(b):
            @pl.loop(0, K)
            def _row(r):
                for j in range(D // 16):
                    sl = pl.ds(j * 16, 16)
                    rows_v[b, r, sl] = rows_v[b, r, sl] * SCALE

        G(0, 0).start()
        G(1, 1).start()
        G(0, 0).wait()
        scale(0)
        S(0, 0).start()

        @pl.loop(0, (n_chunks - 2) // 2)
        def _pair(j):
            c = 1 + 2 * j
            S(c - 1, 0).wait()
            G(c + 1, 0).start()
            G(c, 1).wait()
            scale(1)
            S(c, 1).start()

            S(c, 1).wait()
            G(c + 2, 1).start()
            G(c + 1, 0).wait()
            scale(0)
            S(c + 1, 0).start()

        c_last = n_chunks - 1
        S(c_last - 1, 0).wait()
        G(c_last, 1).wait()
        scale(1)
        S(c_last, 1).start()
        S(c_last, 1).wait()

    return gather_kernel


def kernel(x, table):
    B, S = x.shape
    _, D = table.shape
    idx = x.reshape(B * S).astype(jnp.int32)
    out = _make_gather(B * S, D)(table, idx)
    return out.reshape(B, S, D)


# speedup vs baseline: 2.0844x; 1.0065x over previous
import functools

import jax
import jax.numpy as jnp
from jax import lax
from jax.experimental import pallas as pl
from jax.experimental.pallas import tpu as pltpu
from jax.experimental.pallas import tpu_sc as plsc

N_EMBED = 1024
SCALE = 32.0

_info = plsc.get_sparse_core_info()
_NC, _NS = _info.num_cores, _info.num_subcores
_NW = _NC * _NS


@functools.cache
def _make_gather(B, D):
    per_w = B // _NW
    K = 32
    n_chunks = per_w // K
    mesh = plsc.VectorSubcoreMesh(core_axis_name="c", subcore_axis_name="s")

    @functools.partial(
        pl.kernel,
        mesh=mesh,
        out_type=jax.ShapeDtypeStruct((B, D), jnp.float32),
        scratch_types=[
            pltpu.VMEM((per_w,), jnp.int32),
            pltpu.VMEM((3, K, D), jnp.float32),
            (pltpu.SemaphoreType.DMA,) * 3,
            (pltpu.SemaphoreType.DMA,) * 3,
        ],
    )
    def gather_kernel(table_hbm, idx_hbm, out_hbm, idx_v, rows_v, gsems, ssems):
        wid = lax.axis_index("s") * _NC + lax.axis_index("c")
        base = wid * per_w
        pltpu.sync_copy(idx_hbm.at[pl.ds(base, per_w)], idx_v)

        def G(i, b):
            return pltpu.make_async_copy(
                table_hbm.at[idx_v.at[pl.ds(i * K, K)]], rows_v.at[b], gsems[b])

        def S(i, b):
            return pltpu.make_async_copy(
                rows_v.at[b], out_hbm.at[pl.ds(base + i * K, K)], ssems[b])

        def scale(b):
            @pl.loop(0, K)
            def _row(r):
                for j in range(D // 16):
                    sl = pl.ds(j * 16, 16)
                    rows_v[b, r, sl] = rows_v[b, r, sl] * SCALE

        def chunk(i, b, *, wait_prev_s=True, prefetch=True):
            if wait_prev_s:
                S(i - 1, (b - 1) % 3).wait()
            if prefetch:
                G(i + 2, (b + 2) % 3).start()
            G(i, b).wait()
            scale(b)
            S(i, b).start()

        G(0, 0).start()
        G(1, 1).start()
        G(2, 2).start()
        chunk(0, 0, wait_prev_s=False, prefetch=False)

        @pl.loop(0, (n_chunks - 5) // 3)
        def _triple(j):
            c = 1 + 3 * j
            chunk(c, 1)
            chunk(c + 1, 2)
            chunk(c + 2, 0)

        chunk(n_chunks - 4, 1)
        chunk(n_chunks - 3, 2)
        chunk(n_chunks - 2, 0, prefetch=False)
        chunk(n_chunks - 1, 1, prefetch=False)
        S(n_chunks - 1, 1).wait()

    return gather_kernel


def kernel(x, table):
    B, S = x.shape
    _, D = table.shape
    idx = x.reshape(B * S).astype(jnp.int32)
    out = _make_gather(B * S, D)(table, idx)
    return out.reshape(B, S, D)


# speedup vs baseline: 2.3741x; 1.1390x over previous
import functools

import jax
import jax.numpy as jnp
from jax import lax
from jax.experimental import pallas as pl
from jax.experimental.pallas import tpu as pltpu
from jax.experimental.pallas import tpu_sc as plsc

N_EMBED = 1024
SCALE = 32.0

_info = plsc.get_sparse_core_info()
_NC, _NS = _info.num_cores, _info.num_subcores
_NW = _NC * _NS


@functools.cache
def _make_gather(B, D):
    per_w = B // _NW
    K = 32
    n_chunks = per_w // K
    mesh = plsc.VectorSubcoreMesh(core_axis_name="c", subcore_axis_name="s")

    @functools.partial(
        pl.kernel,
        mesh=mesh,
        out_type=jax.ShapeDtypeStruct((B, D), jnp.float32),
        scratch_types=[
            pltpu.VMEM((per_w,), jnp.int32),
            pltpu.VMEM((3, K, D), jnp.float32),
            (pltpu.SemaphoreType.DMA,) * 3,
            (pltpu.SemaphoreType.DMA,) * 3,
        ],
    )
    def gather_kernel(table_hbm, idx_hbm, out_hbm, idx_v, rows_v, gsems, ssems):
        wid = lax.axis_index("s") * _NC + lax.axis_index("c")
        base = wid * per_w
        pltpu.sync_copy(idx_hbm.at[pl.ds(base, per_w)], idx_v)

        def G(i, b):
            return pltpu.make_async_copy(
                table_hbm.at[idx_v.at[pl.ds(i * K, K)]], rows_v.at[b], gsems[b])

        def S(i, b):
            return pltpu.make_async_copy(
                rows_v.at[b], out_hbm.at[pl.ds(base + i * K, K)], ssems[b])

        def scale(b):
            @functools.partial(plsc.parallel_loop, 0, K, unroll=2)
            def _row(r):
                for j in range(D // 16):
                    sl = pl.ds(j * 16, 16)
                    rows_v[b, r, sl] = rows_v[b, r, sl] * SCALE

        def chunk(i, b, *, wait_prev_s=True, prefetch=True):
            if wait_prev_s:
                S(i - 1, (b - 1) % 3).wait()
            if prefetch:
                G(i + 2, (b + 2) % 3).start()
            G(i, b).wait()
            scale(b)
            S(i, b).start()

        G(0, 0).start()
        G(1, 1).start()
        G(2, 2).start()
        chunk(0, 0, wait_prev_s=False, prefetch=False)

        @pl.loop(0, (n_chunks - 5) // 3)
        def _triple(j):
            c = 1 + 3 * j
            chunk(c, 1)
            chunk(c + 1, 2)
            chunk(c + 2, 0)

        chunk(n_chunks - 4, 1)
        chunk(n_chunks - 3, 2)
        chunk(n_chunks - 2, 0, prefetch=False)
        chunk(n_chunks - 1, 1, prefetch=False)
        S(n_chunks - 1, 1).wait()

    return gather_kernel


def kernel(x, table):
    B, S = x.shape
    _, D = table.shape
    idx = x.reshape(B * S).astype(jnp.int32)
    out = _make_gather(B * S, D)(table, idx)
    return out.reshape(B, S, D)
